# baseline (speedup 1.0000x reference)
import jax
import jax.numpy as jnp
from jax import lax
from jax.experimental import pallas as pl
from jax.experimental.pallas import tpu as pltpu
from jax.experimental.pallas import tpu_sc as plsc

B_, H_, W_, CH = 4, 128, 128, 96
N = B_ * H_ * W_ * CH
TOTAL = N * 4
NC, NS, L = 2, 16, 16

C = 1_572_864
DUM = 65_536
ACC = C + DUM
NPASS = 8
S_TILE = C // NS
Z_TILE = ACC // NS
BLK = 4_096
PER_TILE = N // NS
NBLK = PER_TILE // BLK
DUMMY_MASK = DUM - 1


def _body(upd_hbm, idx_hbm, out_hbm, acc,
          ib0, ib1, vb0, vb1, lb0, lb1, ob0, ob1,
          si0, si1, sc0, sc1):
    idxb = [ib0, ib1]
    valb = [vb0, vb1]
    locb = [lb0, lb1]
    vout = [ob0, ob1]
    sin = [si0, si1]
    ssc = [sc0, sc1]
    core = lax.axis_index("c")
    sub = lax.axis_index("s")
    tile_start = sub * PER_TILE
    zeros16 = jnp.zeros((L,), jnp.float32)
    iota2 = lax.iota(jnp.int32, L) * 2

    def _issue_in(b, par):
        st = tile_start + b * BLK
        pltpu.async_copy(idx_hbm.at[pl.ds(st, BLK)], idxb[par], sin[par])
        pltpu.async_copy(upd_hbm.at[pl.ds(st, BLK)], valb[par], sin[par])

    def _wait_in(b, par):
        st = tile_start + b * BLK
        pltpu.make_async_copy(idx_hbm.at[pl.ds(st, BLK)], idxb[par],
                              sin[par]).wait()
        pltpu.make_async_copy(upd_hbm.at[pl.ds(st, BLK)], valb[par],
                              sin[par]).wait()

    def _wait_scat(par):
        pltpu.make_async_copy(vout[par], acc.at[locb[par]], ssc[par]).wait()

    for p in range(NPASS):
        chunk = 2 * p + core
        base = chunk * C

        _issue_in(0, 0)
        _issue_in(1, 1)

        def _z(i, _):
            ob0[pl.ds(i * L, L)] = zeros16
            return 0
        lax.fori_loop(0, BLK // L, _z, 0)
        for j in range(Z_TILE // BLK):
            pltpu.async_copy(ob0, acc.at[pl.ds(sub * Z_TILE + j * BLK, BLK)],
                             sc0)
        for j in range(Z_TILE // BLK):
            pltpu.make_async_copy(
                ob0, acc.at[pl.ds(sub * Z_TILE + j * BLK, BLK)],
                sc0).wait()
        plsc.subcore_barrier()

        def _blk2(g, _):
            for par in range(2):
                b = g * 2 + par
                _wait_in(b, par)

                @pl.when(b >= 2)
                def _():
                    _wait_scat(par)

                def _vec(i, _):
                    t = i * 256
                    for k in range(8):
                        off = t + k * L
                        vi0 = idxb[par][pl.ds(off, L)]
                        vi1 = idxb[par][pl.ds(off + 128, L)]
                        v32 = valb[par][pl.ds(off, 2 * L)]
                        fa, fb = plsc.unpack(
                            v32, format=plsc.PackFormat.INTERLEAVED)
                        lo0 = vi0 - base
                        in0 = plsc.bitcast(lo0, jnp.uint32) < jnp.uint32(C)
                        du0 = C + (vi0 & DUMMY_MASK)
                        locb[par][pl.ds(off, L)] = jnp.where(in0, lo0, du0)
                        lo1 = vi1 - base
                        in1 = plsc.bitcast(lo1, jnp.uint32) < jnp.uint32(C)
                        du1 = C + (vi1 & DUMMY_MASK)
                        locb[par][pl.ds(off + 128, L)] = jnp.where(
                            in1, lo1, du1)
                        vout[par][pl.ds(off, L)] = fa
                        vout[par][pl.ds(off + 128, L)] = fb
                    return 0
                lax.fori_loop(0, BLK // 256, _vec, 0)

                pltpu.async_copy(vout[par], acc.at[locb[par]], ssc[par],
                                 add=True)

                @pl.when(b + 2 < NBLK)
                def _():
                    _issue_in(b + 2, par)
            return 0
        lax.fori_loop(0, NBLK // 2, _blk2, 0)

        _wait_scat(0)
        _wait_scat(1)
        plsc.subcore_barrier()

        pltpu.sync_copy(acc.at[pl.ds(sub * S_TILE, S_TILE)],
                        out_hbm.at[pl.ds(base + sub * S_TILE, S_TILE)])
        plsc.subcore_barrier()


_scatter = pl.kernel(
    _body,
    out_type=jax.ShapeDtypeStruct((TOTAL,), jnp.float32),
    mesh=plsc.VectorSubcoreMesh(
        core_axis_name="c", subcore_axis_name="s", num_cores=NC,
        num_subcores=NS),
    compiler_params=pltpu.CompilerParams(needs_layout_passes=False),
    scratch_types=[
        pltpu.VMEM_SHARED((ACC,), jnp.float32),
        pltpu.VMEM((BLK,), jnp.int32),
        pltpu.VMEM((BLK,), jnp.int32),
        pltpu.VMEM((BLK,), jnp.bfloat16),
        pltpu.VMEM((BLK,), jnp.bfloat16),
        pltpu.VMEM((BLK,), jnp.int32),
        pltpu.VMEM((BLK,), jnp.int32),
        pltpu.VMEM((BLK,), jnp.float32),
        pltpu.VMEM((BLK,), jnp.float32),
        pltpu.SemaphoreType.DMA,
        pltpu.SemaphoreType.DMA,
        pltpu.SemaphoreType.DMA,
        pltpu.SemaphoreType.DMA,
    ],
)


@jax.jit
def kernel(updates, mask):
    upd = updates.reshape(-1).astype(jnp.bfloat16)
    idx = mask.reshape(-1).astype(jnp.int32)
    out = _scatter(upd, idx)
    return out.reshape(B_, H_ * 2, W_ * 2, CH)


# speedup vs baseline: 1.0001x; 1.0001x over previous
import jax
import jax.numpy as jnp
from jax import lax
from jax.experimental import pallas as pl
from jax.experimental.pallas import tpu as pltpu
from jax.experimental.pallas import tpu_sc as plsc

B_, H_, W_, CH = 4, 128, 128, 96
N = B_ * H_ * W_ * CH
TOTAL = N * 4
NC, NS, L = 2, 16, 16

C = 1_572_864
DUM = 65_536
ACC = C + DUM
NPASS = 8
S_TILE = C // NS
Z_TILE = ACC // NS
BLK = 4_096
PER_TILE = N // NS
NBLK = PER_TILE // BLK
DUMMY_MASK = DUM - 1


def _body(upd_hbm, idx_hbm, out_hbm, acc,
          ib0, ib1, vb0, vb1, vb2, lb0, lb1,
          si0, si1, sv0, sv1, sv2, sc0, sc1):
    idxb = [ib0, ib1]
    valb = [vb0, vb1, vb2]
    locb = [lb0, lb1]
    sin = [si0, si1]
    svin = [sv0, sv1, sv2]
    ssc = [sc0, sc1]
    core = lax.axis_index("c")
    sub = lax.axis_index("s")
    tile_start = sub * PER_TILE
    zeros16 = jnp.zeros((L,), jnp.float32)

    def _issue_idx(b, p2):
        st = tile_start + b * BLK
        pltpu.async_copy(idx_hbm.at[pl.ds(st, BLK)], idxb[p2], sin[p2])

    def _wait_idx(b, p2):
        st = tile_start + b * BLK
        pltpu.make_async_copy(idx_hbm.at[pl.ds(st, BLK)], idxb[p2],
                              sin[p2]).wait()

    def _issue_val(b, p3):
        st = tile_start + b * BLK
        pltpu.async_copy(upd_hbm.at[pl.ds(st, BLK)], valb[p3], svin[p3])

    def _wait_val(b, p3):
        st = tile_start + b * BLK
        pltpu.make_async_copy(upd_hbm.at[pl.ds(st, BLK)], valb[p3],
                              svin[p3]).wait()

    def _wait_scat(p2, p3):
        pltpu.make_async_copy(valb[p3], acc.at[locb[p2]], ssc[p2]).wait()

    for p in range(NPASS):
        chunk = 2 * p + core
        base = chunk * C

        _issue_idx(0, 0)
        _issue_idx(1, 1)
        _issue_val(0, 0)

        def _z(i, _):
            vb2[pl.ds(i * L, L)] = zeros16
            return 0
        lax.fori_loop(0, BLK // L, _z, 0)
        for j in range(Z_TILE // BLK):
            pltpu.async_copy(vb2, acc.at[pl.ds(sub * Z_TILE + j * BLK, BLK)],
                             sc0)
        for j in range(Z_TILE // BLK):
            pltpu.make_async_copy(
                vb2, acc.at[pl.ds(sub * Z_TILE + j * BLK, BLK)],
                sc0).wait()
        plsc.subcore_barrier()

        def _blk6(g, _):
            for k in range(6):
                b6 = g * 6 + k
                p2 = k % 2
                p3 = k % 3

                _wait_idx(b6, p2)

                @pl.when(b6 >= 2)
                def _():
                    _wait_scat(p2, (k + 1) % 3)

                @pl.when(b6 + 1 < NBLK)
                def _():
                    _issue_val(b6 + 1, (k + 1) % 3)

                def _vec(i, _):
                    for u in range(4):
                        off = i * (4 * L) + u * L
                        vi = idxb[p2][pl.ds(off, L)]
                        local = vi - base
                        inr = plsc.bitcast(local, jnp.uint32) < jnp.uint32(C)
                        dummy = C + (vi & DUMMY_MASK)
                        locb[p2][pl.ds(off, L)] = jnp.where(inr, local, dummy)
                    return 0
                lax.fori_loop(0, BLK // (4 * L), _vec, 0)

                _wait_val(b6, p3)
                pltpu.async_copy(valb[p3], acc.at[locb[p2]], ssc[p2],
                                 add=True)

                @pl.when(b6 + 2 < NBLK)
                def _():
                    _issue_idx(b6 + 2, p2)
            return 0
        lax.fori_loop(0, NBLK // 6, _blk6, 0)

        _wait_scat(0, (NBLK - 2) % 3)
        _wait_scat(1, (NBLK - 1) % 3)
        plsc.subcore_barrier()

        pltpu.sync_copy(acc.at[pl.ds(sub * S_TILE, S_TILE)],
                        out_hbm.at[pl.ds(base + sub * S_TILE, S_TILE)])
        plsc.subcore_barrier()


_scatter = pl.kernel(
    _body,
    out_type=jax.ShapeDtypeStruct((TOTAL,), jnp.float32),
    mesh=plsc.VectorSubcoreMesh(
        core_axis_name="c", subcore_axis_name="s", num_cores=NC,
        num_subcores=NS),
    compiler_params=pltpu.CompilerParams(needs_layout_passes=False),
    scratch_types=[
        pltpu.VMEM_SHARED((ACC,), jnp.float32),
        pltpu.VMEM((BLK,), jnp.int32),
        pltpu.VMEM((BLK,), jnp.int32),
        pltpu.VMEM((BLK,), jnp.float32),
        pltpu.VMEM((BLK,), jnp.float32),
        pltpu.VMEM((BLK,), jnp.float32),
        pltpu.VMEM((BLK,), jnp.int32),
        pltpu.VMEM((BLK,), jnp.int32),
        pltpu.SemaphoreType.DMA,
        pltpu.SemaphoreType.DMA,
        pltpu.SemaphoreType.DMA,
        pltpu.SemaphoreType.DMA,
        pltpu.SemaphoreType.DMA,
        pltpu.SemaphoreType.DMA,
        pltpu.SemaphoreType.DMA,
    ],
)


@jax.jit
def kernel(updates, mask):
    upd = updates.reshape(-1)
    idx = mask.reshape(-1).astype(jnp.int32)
    out = _scatter(upd, idx)
    return out.reshape(B_, H_ * 2, W_ * 2, CH)


# speedup vs baseline: 1.0019x; 1.0018x over previous
import jax
import jax.numpy as jnp
from jax import lax
from jax.experimental import pallas as pl
from jax.experimental.pallas import tpu as pltpu
from jax.experimental.pallas import tpu_sc as plsc

B_, H_, W_, CH = 4, 128, 128, 96
N = B_ * H_ * W_ * CH
TOTAL = N * 4
NC, NS, L = 2, 16, 16

C = 1_572_864
DUM = 65_536
ACC = C + DUM
NPASS = 8
S_TILE = C // NS
Z_TILE = ACC // NS
BLK = 2_048
PER_TILE = N // NS
NBLK = PER_TILE // BLK
DUMMY_MASK = DUM - 1


def _body(upd_hbm, idx_hbm, out_hbm, acc,
          ib0, ib1, vb0, vb1, vb2, vb3, lb0, lb1, lb2,
          si0, si1, sv0, sv1, sv2, sv3, sc0, sc1, sc2):
    idxb = [ib0, ib1]
    valb = [vb0, vb1, vb2, vb3]
    locb = [lb0, lb1, lb2]
    sin = [si0, si1]
    svin = [sv0, sv1, sv2, sv3]
    ssc = [sc0, sc1, sc2]
    core = lax.axis_index("c")
    sub = lax.axis_index("s")
    tile_start = sub * PER_TILE
    zeros16 = jnp.zeros((L,), jnp.float32)

    def _issue_idx(b, p2):
        st = tile_start + b * BLK
        pltpu.async_copy(idx_hbm.at[pl.ds(st, BLK)], idxb[p2], sin[p2])

    def _wait_idx(b, p2):
        st = tile_start + b * BLK
        pltpu.make_async_copy(idx_hbm.at[pl.ds(st, BLK)], idxb[p2],
                              sin[p2]).wait()

    def _issue_val(b, p3):
        st = tile_start + b * BLK
        pltpu.async_copy(upd_hbm.at[pl.ds(st, BLK)], valb[p3], svin[p3])

    def _wait_val(b, p3):
        st = tile_start + b * BLK
        pltpu.make_async_copy(upd_hbm.at[pl.ds(st, BLK)], valb[p3],
                              svin[p3]).wait()

    def _wait_scat(pl3, pv4):
        pltpu.make_async_copy(valb[pv4], acc.at[locb[pl3]], ssc[pl3]).wait()

    for p in range(NPASS):
        chunk = 2 * p + core
        base = chunk * C

        _issue_idx(0, 0)
        _issue_idx(1, 1)
        _issue_val(0, 0)

        def _z(i, _):
            vb3[pl.ds(i * L, L)] = zeros16
            return 0
        lax.fori_loop(0, BLK // L, _z, 0)
        for j in range(Z_TILE // BLK):
            pltpu.async_copy(vb3, acc.at[pl.ds(sub * Z_TILE + j * BLK, BLK)],
                             sc0)
        for j in range(Z_TILE // BLK):
            pltpu.make_async_copy(
                vb3, acc.at[pl.ds(sub * Z_TILE + j * BLK, BLK)],
                sc0).wait()
        plsc.subcore_barrier()

        def _blk12(g, _):
            for k in range(12):
                b = g * 12 + k
                p2 = k % 2
                p3 = k % 3
                p4 = k % 4

                _wait_idx(b, p2)

                @pl.when(b >= 3)
                def _():
                    _wait_scat(p3, (k + 1) % 4)

                @pl.when(b + 1 < NBLK)
                def _():
                    _issue_val(b + 1, (k + 1) % 4)

                def _vec(i, _):
                    for u in range(4):
                        off = i * (4 * L) + u * L
                        vi = idxb[p2][pl.ds(off, L)]
                        local = vi - base
                        inr = plsc.bitcast(local, jnp.uint32) < jnp.uint32(C)
                        dummy = C + (vi & DUMMY_MASK)
                        locb[p3][pl.ds(off, L)] = jnp.where(inr, local, dummy)
                    return 0
                lax.fori_loop(0, BLK // (4 * L), _vec, 0)

                _wait_val(b, p4)
                pltpu.async_copy(valb[p4], acc.at[locb[p3]], ssc[p3],
                                 add=True)

                @pl.when(b + 2 < NBLK)
                def _():
                    _issue_idx(b + 2, p2)
            return 0
        lax.fori_loop(0, NBLK // 12, _blk12, 0)

        _wait_scat((NBLK - 3) % 3, (NBLK - 3) % 4)
        _wait_scat((NBLK - 2) % 3, (NBLK - 2) % 4)
        _wait_scat((NBLK - 1) % 3, (NBLK - 1) % 4)
        plsc.subcore_barrier()

        pltpu.sync_copy(acc.at[pl.ds(sub * S_TILE, S_TILE)],
                        out_hbm.at[pl.ds(base + sub * S_TILE, S_TILE)])
        plsc.subcore_barrier()


_scatter = pl.kernel(
    _body,
    out_type=jax.ShapeDtypeStruct((TOTAL,), jnp.float32),
    mesh=plsc.VectorSubcoreMesh(
        core_axis_name="c", subcore_axis_name="s", num_cores=NC,
        num_subcores=NS),
    compiler_params=pltpu.CompilerParams(needs_layout_passes=False),
    scratch_types=[
        pltpu.VMEM_SHARED((ACC,), jnp.float32),
        pltpu.VMEM((BLK,), jnp.int32),
        pltpu.VMEM((BLK,), jnp.int32),
        pltpu.VMEM((BLK,), jnp.float32),
        pltpu.VMEM((BLK,), jnp.float32),
        pltpu.VMEM((BLK,), jnp.float32),
        pltpu.VMEM((BLK,), jnp.float32),
        pltpu.VMEM((BLK,), jnp.int32),
        pltpu.VMEM((BLK,), jnp.int32),
        pltpu.VMEM((BLK,), jnp.int32),
        pltpu.SemaphoreType.DMA,
        pltpu.SemaphoreType.DMA,
        pltpu.SemaphoreType.DMA,
        pltpu.SemaphoreType.DMA,
        pltpu.SemaphoreType.DMA,
        pltpu.SemaphoreType.DMA,
        pltpu.SemaphoreType.DMA,
        pltpu.SemaphoreType.DMA,
        pltpu.SemaphoreType.DMA,
    ],
)


@jax.jit
def kernel(updates, mask):
    upd = updates.reshape(-1)
    idx = mask.reshape(-1).astype(jnp.int32)
    out = _scatter(upd, idx)
    return out.reshape(B_, H_ * 2, W_ * 2, CH)
